# Initial kernel scaffold; baseline (speedup 1.0000x reference)
import jax
import jax.numpy as jnp
from jax.experimental import pallas as pl


def kernel(x, lut):
    raise NotImplementedError("write your pallas kernel here")


# speedup vs baseline: 5.8563x; 5.8563x over previous
import functools

import jax
import jax.numpy as jnp
from jax import lax
from jax.experimental import pallas as pl
from jax.experimental.pallas import tpu as pltpu
from jax.experimental.pallas import tpu_sc as plsc

NUM_COLORS = 256
B, T, F = 64, 1024, 128
N = T * F
NW = 32
BPW = B // NW
CH = 8192
NCH = N // CH
VPC = CH // 16
L = 16

_mesh = plsc.VectorSubcoreMesh(core_axis_name="c", subcore_axis_name="s")


@functools.partial(
    pl.kernel,
    mesh=_mesh,
    out_type=jax.ShapeDtypeStruct((B * N * 3,), jnp.float32),
    scratch_types=[
        pltpu.VMEM((NUM_COLORS * 3,), jnp.float32),
        pltpu.VMEM((CH,), jnp.float32),
        pltpu.VMEM((CH * 3,), jnp.float32),
        pltpu.VMEM((L,), jnp.float32),
    ],
    compiler_params=pltpu.CompilerParams(needs_layout_passes=False),
)
def _mel_to_rgb(x_hbm, lut_hbm, out_hbm, lut_v, xbuf, obuf, red_v):
    cid = lax.axis_index("c")
    sid = lax.axis_index("s")
    wid = sid * 2 + cid

    pltpu.sync_copy(lut_hbm, lut_v)
    lane = lax.iota(jnp.int32, 16)
    off3 = lane * 3

    def lane_reduce(v, op):
        for k in (8, 4, 2, 1):
            red_v[...] = v
            v = op(v, plsc.load_gather(red_v, [lane ^ k]))
        return v

    for j in range(BPW):
        b = wid * BPW + j
        base = b * N

        def chunk_a(ci, carry):
            mnv, mxv = carry
            pltpu.sync_copy(x_hbm.at[pl.ds(base + ci * CH, CH)], xbuf)

            def vec_a(vi, c2):
                mn2, mx2 = c2
                xv = xbuf[pl.ds(vi * L, L)]
                return jnp.minimum(mn2, xv), jnp.maximum(mx2, xv)

            return lax.fori_loop(0, VPC, vec_a, (mnv, mxv), unroll=8)

        big = jnp.full((L,), jnp.inf, jnp.float32)
        mnv, mxv = lax.fori_loop(0, NCH, chunk_a, (big, -big))
        mnv = lane_reduce(mnv, jnp.minimum)
        mxv = lane_reduce(mxv, jnp.maximum)

        scale_v = (NUM_COLORS - 1) / (mxv - mnv + 1e-6)
        off_v = 0.5 - mnv * scale_v

        def chunk_b(ci, _):
            pltpu.sync_copy(x_hbm.at[pl.ds(base + ci * CH, CH)], xbuf)

            def vec_b(vi, __):
                xv = xbuf[pl.ds(vi * L, L)]
                t = xv * scale_v + off_v
                idx = t.astype(jnp.int32)
                idx = jnp.minimum(jnp.maximum(idx, 0), NUM_COLORS - 1)
                i3 = idx * 3
                r = plsc.load_gather(lut_v, [i3])
                g = plsc.load_gather(lut_v, [i3 + 1])
                bl = plsc.load_gather(lut_v, [i3 + 2])
                pos = vi * (3 * L) + off3
                plsc.store_scatter(obuf, [pos], r)
                plsc.store_scatter(obuf, [pos + 1], g)
                plsc.store_scatter(obuf, [pos + 2], bl)
                return 0

            lax.fori_loop(0, VPC, vec_b, 0, unroll=4)
            pltpu.sync_copy(obuf, out_hbm.at[pl.ds(base * 3 + ci * CH * 3, CH * 3)])
            return 0

        lax.fori_loop(0, NCH, chunk_b, 0)


def kernel(x, lut):
    out = _mel_to_rgb(x.reshape(-1), lut.reshape(-1))
    return out.reshape(B, T, F, 3)


# speedup vs baseline: 49.6376x; 8.4759x over previous
import functools

import jax
import jax.numpy as jnp
from jax import lax
from jax.experimental import pallas as pl
from jax.experimental.pallas import tpu as pltpu
from jax.experimental.pallas import tpu_sc as plsc

NUM_COLORS = 256
B, T, F = 64, 1024, 128
N = T * F
NW = 32
BPW = B // NW
TT = 64
CH = TT * F
NCH = T // TT
VPC = CH // 16
L = 16

_mesh = plsc.VectorSubcoreMesh(core_axis_name="c", subcore_axis_name="s")


@functools.partial(
    pl.kernel,
    mesh=_mesh,
    out_type=jax.ShapeDtypeStruct((B, T, 8, F), jnp.float32),
    scratch_types=[
        pltpu.VMEM((NUM_COLORS * 3,), jnp.float32),
        pltpu.VMEM((CH,), jnp.float32),
        pltpu.VMEM((TT, 4, F), jnp.float32),
        pltpu.VMEM((L,), jnp.float32),
    ],
    compiler_params=pltpu.CompilerParams(
        needs_layout_passes=False, disable_bounds_checks=True),
)
def _mel_to_rgb(x_hbm, lut_hbm, out_hbm, lut_v, xbuf, obuf, red_v):
    cid = lax.axis_index("c")
    sid = lax.axis_index("s")
    wid = sid * 2 + cid

    pltpu.sync_copy(lut_hbm, lut_v)
    lane = lax.iota(jnp.int32, 16)

    def lane_reduce(v, op):
        for k in (8, 4, 2, 1):
            red_v[...] = v
            v = op(v, plsc.load_gather(red_v, [lane ^ k]))
        return v

    for j in range(BPW):
        b = wid * BPW + j
        base = b * N

        def chunk_a(ci, carry):
            mnv, mxv = carry
            pltpu.sync_copy(x_hbm.at[pl.ds(base + ci * CH, CH)], xbuf)

            def vec_a(vi, c2):
                mn2, mx2 = c2
                xv = xbuf[pl.ds(vi * L, L)]
                return jnp.minimum(mn2, xv), jnp.maximum(mx2, xv)

            return lax.fori_loop(0, VPC, vec_a, (mnv, mxv), unroll=8)

        bigv = jnp.full((L,), jnp.inf, jnp.float32)
        mnv, mxv = lax.fori_loop(0, NCH, chunk_a, (bigv, -bigv))
        mnv = lane_reduce(mnv, jnp.minimum)
        mxv = lane_reduce(mxv, jnp.maximum)

        scale_v = (NUM_COLORS - 1) / (mxv - mnv + 1e-6)
        off_v = 0.5 - mnv * scale_v

        def chunk_b(ci, _):
            pltpu.sync_copy(x_hbm.at[pl.ds(base + ci * CH, CH)], xbuf)

            def row_b(trow, __):
                for k in range(F // L):
                    f0 = k * L
                    xv = xbuf[pl.ds(trow * F + f0, L)]
                    t = xv * scale_v + off_v
                    idx = t.astype(jnp.int32)
                    idx = jnp.minimum(jnp.maximum(idx, 0), NUM_COLORS - 1)
                    r = plsc.load_gather(lut_v, [idx])
                    g = plsc.load_gather(lut_v, [idx + NUM_COLORS])
                    bl = plsc.load_gather(lut_v, [idx + 2 * NUM_COLORS])
                    obuf[trow, 0, pl.ds(f0, L)] = r
                    obuf[trow, 1, pl.ds(f0, L)] = g
                    obuf[trow, 2, pl.ds(f0, L)] = bl
                return 0

            lax.fori_loop(0, TT, row_b, 0)
            pltpu.sync_copy(obuf.at[:, pl.ds(0, 3), :],
                            out_hbm.at[b, pl.ds(ci * TT, TT), pl.ds(0, 3)])
            return 0

        lax.fori_loop(0, NCH, chunk_b, 0)


def kernel(x, lut):
    lut_planar = lut.T.reshape(-1)
    out = _mel_to_rgb(x.reshape(-1), lut_planar)
    return out.transpose(0, 1, 3, 2)[..., :3]


# speedup vs baseline: 83.5443x; 1.6831x over previous
import functools

import jax
import jax.numpy as jnp
from jax import lax
from jax.experimental import pallas as pl
from jax.experimental.pallas import tpu as pltpu
from jax.experimental.pallas import tpu_sc as plsc

NUM_COLORS = 256
B, T, F = 64, 1024, 128
N = T * F
NW = 32
BPW = B // NW
TT = 64
CH = TT * F
NCH = T // TT
VPC = CH // 16
L = 16

_mesh = plsc.VectorSubcoreMesh(core_axis_name="c", subcore_axis_name="s")


@functools.partial(
    pl.kernel,
    mesh=_mesh,
    out_type=jax.ShapeDtypeStruct((B, T, 8, F), jnp.float32),
    scratch_types=[
        pltpu.VMEM((NUM_COLORS * 3,), jnp.float32),
        pltpu.VMEM((CH,), jnp.float32),
        pltpu.VMEM((TT, 4, F), jnp.float32),
        pltpu.VMEM((L,), jnp.float32),
    ],
    compiler_params=pltpu.CompilerParams(
        needs_layout_passes=False, disable_bounds_checks=True),
)
def _mel_to_rgb(x_hbm, lut_hbm, out_hbm, lut_v, xbuf, obuf, red_v):
    cid = lax.axis_index("c")
    sid = lax.axis_index("s")
    wid = sid * 2 + cid

    pltpu.sync_copy(lut_hbm, lut_v)
    lane = lax.iota(jnp.int32, 16)

    def lane_reduce(v, op):
        for k in (8, 4, 2, 1):
            red_v[...] = v
            v = op(v, plsc.load_gather(red_v, [lane ^ k]))
        return v

    for j in range(BPW):
        b = wid * BPW + j
        base = b * N

        def chunk_a(ci, carry):
            mnv, mxv = carry
            pltpu.sync_copy(x_hbm.at[pl.ds(base + ci * CH, CH)], xbuf)

            def vec_a(vi, c2):
                mn2, mx2 = c2
                xv = xbuf[pl.ds(vi * L, L)]
                return jnp.minimum(mn2, xv), jnp.maximum(mx2, xv)

            return lax.fori_loop(0, VPC, vec_a, (mnv, mxv), unroll=8)

        bigv = jnp.full((L,), jnp.inf, jnp.float32)
        mnv, mxv = lax.fori_loop(0, NCH, chunk_a, (bigv, -bigv))
        mnv = lane_reduce(mnv, jnp.minimum)
        mxv = lane_reduce(mxv, jnp.maximum)

        scale_v = (NUM_COLORS - 1) / (mxv - mnv + 1e-6)
        off_v = 0.5 - mnv * scale_v

        def chunk_b(ci, _):
            pltpu.sync_copy(x_hbm.at[pl.ds(base + ci * CH, CH)], xbuf)

            @plsc.parallel_loop(0, TT, 1, unroll=2)
            def row_b(trow):
                for k in range(F // L):
                    f0 = k * L
                    xv = xbuf[pl.ds(trow * F + f0, L)]
                    t = xv * scale_v + off_v
                    idx = t.astype(jnp.int32)
                    idx = jnp.minimum(jnp.maximum(idx, 0), NUM_COLORS - 1)
                    r = plsc.load_gather(lut_v, [idx])
                    g = plsc.load_gather(lut_v, [idx + NUM_COLORS])
                    bl = plsc.load_gather(lut_v, [idx + 2 * NUM_COLORS])
                    obuf[trow, 0, pl.ds(f0, L)] = r
                    obuf[trow, 1, pl.ds(f0, L)] = g
                    obuf[trow, 2, pl.ds(f0, L)] = bl
            pltpu.sync_copy(obuf.at[:, pl.ds(0, 3), :],
                            out_hbm.at[b, pl.ds(ci * TT, TT), pl.ds(0, 3)])
            return 0

        lax.fori_loop(0, NCH, chunk_b, 0)


def kernel(x, lut):
    lut_planar = lut.T.reshape(-1)
    out = _mel_to_rgb(x.reshape(-1), lut_planar)
    return out.transpose(0, 1, 3, 2)[..., :3]


# speedup vs baseline: 100.9877x; 1.2088x over previous
import functools

import jax
import jax.numpy as jnp
from jax import lax
from jax.experimental import pallas as pl
from jax.experimental.pallas import tpu as pltpu
from jax.experimental.pallas import tpu_sc as plsc

NUM_COLORS = 256
B, T, F = 64, 1024, 128
N = T * F
NW = 32
BPW = B // NW
TT = 64
CH = TT * F
NCH = T // TT
VPC = CH // 16
L = 16

_mesh = plsc.VectorSubcoreMesh(core_axis_name="c", subcore_axis_name="s")


@functools.partial(
    pl.kernel,
    mesh=_mesh,
    out_type=jax.ShapeDtypeStruct((B, T, 8, F), jnp.float32),
    scratch_types=[
        pltpu.VMEM((NUM_COLORS * 3,), jnp.float32),
        pltpu.VMEM((CH,), jnp.float32),
        pltpu.VMEM((CH,), jnp.float32),
        pltpu.VMEM((TT, 4, F), jnp.float32),
        pltpu.VMEM((TT, 4, F), jnp.float32),
        pltpu.VMEM((L,), jnp.float32),
        pltpu.SemaphoreType.DMA,
        pltpu.SemaphoreType.DMA,
        pltpu.SemaphoreType.DMA,
        pltpu.SemaphoreType.DMA,
    ],
    compiler_params=pltpu.CompilerParams(
        needs_layout_passes=False, disable_bounds_checks=True),
)
def _mel_to_rgb(x_hbm, lut_hbm, out_hbm, lut_v, xb0, xb1, ob0, ob1, red_v,
                si0, si1, so0, so1):
    cid = lax.axis_index("c")
    sid = lax.axis_index("s")
    wid = sid * 2 + cid

    xbs, obs = (xb0, xb1), (ob0, ob1)
    sis, sos = (si0, si1), (so0, so1)

    pltpu.sync_copy(lut_hbm, lut_v)
    lane = lax.iota(jnp.int32, 16)

    def lane_reduce(v, op):
        for k in (8, 4, 2, 1):
            red_v[...] = v
            v = op(v, plsc.load_gather(red_v, [lane ^ k]))
        return v

    def compute_a(xbuf, accs):
        def vec_a(vi, c2):
            mn0, mx0, mn1, mx1 = c2
            xv0 = xbuf[pl.ds(vi * (2 * L), L)]
            xv1 = xbuf[pl.ds(vi * (2 * L) + L, L)]
            return (jnp.minimum(mn0, xv0), jnp.maximum(mx0, xv0),
                    jnp.minimum(mn1, xv1), jnp.maximum(mx1, xv1))

        return lax.fori_loop(0, VPC // 2, vec_a, accs, unroll=4)

    for j in range(BPW):
        b = wid * BPW + j
        base = b * N

        def start_in(ci, k):
            pltpu.async_copy(
                x_hbm.at[pl.ds(base + ci * CH, CH)], xbs[k], sis[k])

        def wait_in(k):
            pltpu.make_async_copy(
                x_hbm.at[pl.ds(base, CH)], xbs[k], sis[k]).wait()

        def start_out(ci, k):
            pltpu.async_copy(
                obs[k].at[:, pl.ds(0, 3), :],
                out_hbm.at[b, pl.ds(ci * TT, TT), pl.ds(0, 3)], sos[k])

        def wait_out(k):
            pltpu.make_async_copy(
                obs[k].at[:, pl.ds(0, 3), :],
                out_hbm.at[b, pl.ds(0, TT), pl.ds(0, 3)], sos[k]).wait()

        bigv = jnp.full((L,), jnp.inf, jnp.float32)
        start_in(0, 0)
        start_in(1, 1)

        def pair_a(g, accs):
            c0 = 2 * g
            for k in range(2):
                wait_in(k)
                accs = compute_a(xbs[k], accs)

                @pl.when(c0 + 2 + k < NCH)
                def _():
                    start_in(c0 + 2 + k, k)
            return accs

        accs = lax.fori_loop(0, NCH // 2, pair_a, (bigv, -bigv, bigv, -bigv))

        mnv = lane_reduce(jnp.minimum(accs[0], accs[2]), jnp.minimum)
        mxv = lane_reduce(jnp.maximum(accs[1], accs[3]), jnp.maximum)

        scale_v = (NUM_COLORS - 1) / (mxv - mnv + 1e-6)
        off_v = 0.5 - mnv * scale_v

        start_in(0, 0)
        start_in(1, 1)

        def pair_b(g, _):
            c0 = 2 * g
            for k in range(2):
                wait_in(k)

                @pl.when(c0 + k >= 2)
                def _():
                    wait_out(k)

                xbuf, obuf = xbs[k], obs[k]

                @plsc.parallel_loop(0, TT, 1, unroll=2)
                def row_b(trow):
                    for kk in range(F // L):
                        f0 = kk * L
                        xv = xbuf[pl.ds(trow * F + f0, L)]
                        t = xv * scale_v + off_v
                        idx = t.astype(jnp.int32)
                        idx = jnp.minimum(jnp.maximum(idx, 0), NUM_COLORS - 1)
                        r = plsc.load_gather(lut_v, [idx])
                        g2 = plsc.load_gather(lut_v, [idx + NUM_COLORS])
                        bl = plsc.load_gather(lut_v, [idx + 2 * NUM_COLORS])
                        obuf[trow, 0, pl.ds(f0, L)] = r
                        obuf[trow, 1, pl.ds(f0, L)] = g2
                        obuf[trow, 2, pl.ds(f0, L)] = bl

                start_out(c0 + k, k)

                @pl.when(c0 + 2 + k < NCH)
                def _():
                    start_in(c0 + 2 + k, k)
            return 0

        lax.fori_loop(0, NCH // 2, pair_b, 0)
        wait_out(0)
        wait_out(1)


def kernel(x, lut):
    lut_planar = lut.T.reshape(-1)
    out = _mel_to_rgb(x.reshape(-1), lut_planar)
    return out.transpose(0, 1, 3, 2)[..., :3]


# speedup vs baseline: 166.2318x; 1.6461x over previous
import functools

import jax
import jax.numpy as jnp
from jax import lax
from jax.experimental import pallas as pl
from jax.experimental.pallas import tpu as pltpu
from jax.experimental.pallas import tpu_sc as plsc

NUM_COLORS = 256
B, T, F = 64, 1024, 128
N = T * F
NW = 32
BPW = B // NW
TT = 64
CH = TT * F
NCH = T // TT
VPC = CH // 16
L = 16

_mesh = plsc.VectorSubcoreMesh(core_axis_name="c", subcore_axis_name="s")


@functools.partial(
    pl.kernel,
    mesh=_mesh,
    out_type=jax.ShapeDtypeStruct((B, T, 3, F), jnp.float32),
    scratch_types=[
        pltpu.VMEM((NUM_COLORS * 3,), jnp.float32),
        pltpu.VMEM((CH,), jnp.float32),
        pltpu.VMEM((CH,), jnp.float32),
        pltpu.VMEM((TT, 4, F), jnp.float32),
        pltpu.VMEM((TT, 4, F), jnp.float32),
        pltpu.VMEM((L,), jnp.float32),
        pltpu.SemaphoreType.DMA,
        pltpu.SemaphoreType.DMA,
        pltpu.SemaphoreType.DMA,
        pltpu.SemaphoreType.DMA,
    ],
    compiler_params=pltpu.CompilerParams(
        needs_layout_passes=False, disable_bounds_checks=True),
)
def _mel_to_rgb(x_hbm, lut_hbm, out_hbm, lut_v, xb0, xb1, ob0, ob1, red_v,
                si0, si1, so0, so1):
    cid = lax.axis_index("c")
    sid = lax.axis_index("s")
    wid = sid * 2 + cid

    xbs, obs = (xb0, xb1), (ob0, ob1)
    sis, sos = (si0, si1), (so0, so1)

    pltpu.sync_copy(lut_hbm, lut_v)
    lane = lax.iota(jnp.int32, 16)

    def lane_reduce(v, op):
        for k in (8, 4, 2, 1):
            red_v[...] = v
            v = op(v, plsc.load_gather(red_v, [lane ^ k]))
        return v

    def compute_a(xbuf, accs):
        def vec_a(vi, c2):
            mn0, mx0, mn1, mx1 = c2
            xv0 = xbuf[pl.ds(vi * (2 * L), L)]
            xv1 = xbuf[pl.ds(vi * (2 * L) + L, L)]
            return (jnp.minimum(mn0, xv0), jnp.maximum(mx0, xv0),
                    jnp.minimum(mn1, xv1), jnp.maximum(mx1, xv1))

        return lax.fori_loop(0, VPC // 2, vec_a, accs, unroll=4)

    for j in range(BPW):
        b = wid * BPW + j
        base = b * N

        def start_in(ci, k):
            pltpu.async_copy(
                x_hbm.at[pl.ds(base + ci * CH, CH)], xbs[k], sis[k])

        def wait_in(k):
            pltpu.make_async_copy(
                x_hbm.at[pl.ds(base, CH)], xbs[k], sis[k]).wait()

        def start_out(ci, k):
            pltpu.async_copy(
                obs[k].at[:, pl.ds(0, 3), :],
                out_hbm.at[b, pl.ds(ci * TT, TT), pl.ds(0, 3)], sos[k])

        def wait_out(k):
            pltpu.make_async_copy(
                obs[k].at[:, pl.ds(0, 3), :],
                out_hbm.at[b, pl.ds(0, TT), pl.ds(0, 3)], sos[k]).wait()

        bigv = jnp.full((L,), jnp.inf, jnp.float32)
        start_in(0, 0)
        start_in(1, 1)

        def pair_a(g, accs):
            c0 = 2 * g
            for k in range(2):
                wait_in(k)
                accs = compute_a(xbs[k], accs)

                @pl.when(c0 + 2 + k < NCH)
                def _():
                    start_in(c0 + 2 + k, k)
            return accs

        accs = lax.fori_loop(0, NCH // 2, pair_a, (bigv, -bigv, bigv, -bigv))

        mnv = lane_reduce(jnp.minimum(accs[0], accs[2]), jnp.minimum)
        mxv = lane_reduce(jnp.maximum(accs[1], accs[3]), jnp.maximum)

        scale_v = (NUM_COLORS - 1) / (mxv - mnv + 1e-6)
        off_v = 0.5 - mnv * scale_v

        start_in(0, 0)
        start_in(1, 1)

        def pair_b(g, _):
            c0 = 2 * g
            for k in range(2):
                wait_in(k)

                @pl.when(c0 + k >= 2)
                def _():
                    wait_out(k)

                xbuf, obuf = xbs[k], obs[k]

                @plsc.parallel_loop(0, TT, 1, unroll=2)
                def row_b(trow):
                    for kk in range(F // L):
                        f0 = kk * L
                        xv = xbuf[pl.ds(trow * F + f0, L)]
                        t = xv * scale_v + off_v
                        idx = t.astype(jnp.int32)
                        idx = jnp.minimum(jnp.maximum(idx, 0), NUM_COLORS - 1)
                        r = plsc.load_gather(lut_v, [idx])
                        g2 = plsc.load_gather(lut_v, [idx + NUM_COLORS])
                        bl = plsc.load_gather(lut_v, [idx + 2 * NUM_COLORS])
                        obuf[trow, 0, pl.ds(f0, L)] = r
                        obuf[trow, 1, pl.ds(f0, L)] = g2
                        obuf[trow, 2, pl.ds(f0, L)] = bl

                start_out(c0 + k, k)

                @pl.when(c0 + 2 + k < NCH)
                def _():
                    start_in(c0 + 2 + k, k)
            return 0

        lax.fori_loop(0, NCH // 2, pair_b, 0)
        wait_out(0)
        wait_out(1)


def kernel(x, lut):
    lut_planar = lut.T.reshape(-1)
    out = _mel_to_rgb(x.reshape(-1), lut_planar)
    return out.transpose(0, 1, 3, 2)


# speedup vs baseline: 166.6777x; 1.0027x over previous
import functools

import jax
import jax.numpy as jnp
from jax import lax
from jax.experimental import pallas as pl
from jax.experimental.pallas import tpu as pltpu
from jax.experimental.pallas import tpu_sc as plsc

NUM_COLORS = 256
B, T, F = 64, 1024, 128
N = T * F
NW = 32
BPW = B // NW
TT = 64
CH = TT * F
NCH = T // TT
VPC = CH // 16
L = 16

_mesh = plsc.VectorSubcoreMesh(core_axis_name="c", subcore_axis_name="s")


@functools.partial(
    pl.kernel,
    mesh=_mesh,
    out_type=jax.ShapeDtypeStruct((B, T, 3, F), jnp.float32),
    scratch_types=[
        pltpu.VMEM((NUM_COLORS * 3,), jnp.float32),
        pltpu.VMEM((CH,), jnp.float32),
        pltpu.VMEM((CH,), jnp.float32),
        pltpu.VMEM((TT, 4, F), jnp.float32),
        pltpu.VMEM((TT, 4, F), jnp.float32),
        pltpu.VMEM((L,), jnp.float32),
        pltpu.SemaphoreType.DMA,
        pltpu.SemaphoreType.DMA,
        pltpu.SemaphoreType.DMA,
        pltpu.SemaphoreType.DMA,
    ],
    compiler_params=pltpu.CompilerParams(
        needs_layout_passes=False, disable_bounds_checks=True),
)
def _mel_to_rgb(x_hbm, lut_hbm, out_hbm, lut_v, xb0, xb1, ob0, ob1, red_v,
                si0, si1, so0, so1):
    cid = lax.axis_index("c")
    sid = lax.axis_index("s")
    wid = sid * 2 + cid

    xbs, obs = (xb0, xb1), (ob0, ob1)
    sis, sos = (si0, si1), (so0, so1)

    pltpu.sync_copy(lut_hbm, lut_v)
    lane = lax.iota(jnp.int32, 16)

    def lane_reduce(v, op):
        for k in (8, 4, 2, 1):
            red_v[...] = v
            v = op(v, plsc.load_gather(red_v, [lane ^ k]))
        return v

    def compute_a(xbuf, accs):
        def vec_a(vi, c2):
            mn0, mx0, mn1, mx1 = c2
            xv0 = xbuf[pl.ds(vi * (2 * L), L)]
            xv1 = xbuf[pl.ds(vi * (2 * L) + L, L)]
            return (jnp.minimum(mn0, xv0), jnp.maximum(mx0, xv0),
                    jnp.minimum(mn1, xv1), jnp.maximum(mx1, xv1))

        return lax.fori_loop(0, VPC // 2, vec_a, accs, unroll=4)

    for j in range(BPW):
        b = wid * BPW + j
        base = b * N

        def start_in(ci, k):
            pltpu.async_copy(
                x_hbm.at[pl.ds(base + ci * CH, CH)], xbs[k], sis[k])

        def wait_in(k):
            pltpu.make_async_copy(
                x_hbm.at[pl.ds(base, CH)], xbs[k], sis[k]).wait()

        def start_out(ci, k):
            pltpu.async_copy(
                obs[k].at[:, pl.ds(0, 3), :],
                out_hbm.at[b, pl.ds(ci * TT, TT), pl.ds(0, 3)], sos[k])

        def wait_out(k):
            pltpu.make_async_copy(
                obs[k].at[:, pl.ds(0, 3), :],
                out_hbm.at[b, pl.ds(0, TT), pl.ds(0, 3)], sos[k]).wait()

        bigv = jnp.full((L,), jnp.inf, jnp.float32)
        start_in(0, 0)
        start_in(1, 1)

        def pair_a(g, accs):
            c0 = 2 * g
            for k in range(2):
                wait_in(k)
                accs = compute_a(xbs[k], accs)

                @pl.when(c0 + 2 + k < NCH)
                def _():
                    start_in(c0 + 2 + k, k)
            return accs

        accs = lax.fori_loop(0, NCH // 2, pair_a, (bigv, -bigv, bigv, -bigv))

        mnv = lane_reduce(jnp.minimum(accs[0], accs[2]), jnp.minimum)
        mxv = lane_reduce(jnp.maximum(accs[1], accs[3]), jnp.maximum)

        scale_v = (NUM_COLORS - 1) / (mxv - mnv + 1e-6)
        off_v = 0.5 - mnv * scale_v

        start_in(0, 0)
        start_in(1, 1)

        def pair_b(g, _):
            c0 = 2 * g
            for k in range(2):
                wait_in(k)

                @pl.when(c0 + k >= 2)
                def _():
                    wait_out(k)

                xbuf, obuf = xbs[k], obs[k]

                @plsc.parallel_loop(0, TT, 1, unroll=4)
                def row_b(trow):
                    for kk in range(F // L):
                        f0 = kk * L
                        xv = xbuf[pl.ds(trow * F + f0, L)]
                        t = xv * scale_v + off_v
                        idx = t.astype(jnp.int32)
                        idx = jnp.minimum(jnp.maximum(idx, 0), NUM_COLORS - 1)
                        r = plsc.load_gather(lut_v, [idx])
                        g2 = plsc.load_gather(lut_v, [idx + NUM_COLORS])
                        bl = plsc.load_gather(lut_v, [idx + 2 * NUM_COLORS])
                        obuf[trow, 0, pl.ds(f0, L)] = r
                        obuf[trow, 1, pl.ds(f0, L)] = g2
                        obuf[trow, 2, pl.ds(f0, L)] = bl

                start_out(c0 + k, k)

                @pl.when(c0 + 2 + k < NCH)
                def _():
                    start_in(c0 + 2 + k, k)
            return 0

        lax.fori_loop(0, NCH // 2, pair_b, 0)
        wait_out(0)
        wait_out(1)


def kernel(x, lut):
    lut_planar = lut.T.reshape(-1)
    out = _mel_to_rgb(x.reshape(-1), lut_planar)
    return out.transpose(0, 1, 3, 2)
